# baseline (speedup 1.0000x reference)
import functools

import jax
import jax.numpy as jnp
from jax.experimental import pallas as pl
from jax.experimental.pallas import tpu as pltpu

HIDDEN = 64
HEADS = 4
OUT_CH = HIDDEN // HEADS
T_FUTURE = 14
T = 12

GRU_BLOCK = 400


def _gru_head_body(hseq_ref, wih0_ref, whh0_ref, bih0_ref, bhh0_ref,
                   wih1_ref, whh1_ref, bih1_ref, bhh1_ref,
                   headw_ref, headb_ref, out_ref):
    R = hseq_ref.shape[0]
    x = hseq_ref[:]

    def run_layer(xs, wih, whh, bih, bhh):
        gi = jax.lax.dot_general(
            xs.reshape(R * T, HIDDEN), wih,
            (((1,), (1,)), ((), ())),
            preferred_element_type=jnp.float32) + bih[None, :]
        gi = gi.reshape(R, T, 3 * HIDDEN)
        h = jnp.zeros((R, HIDDEN), jnp.float32)
        ys = []
        for t in range(T):
            gh = jax.lax.dot_general(
                h, whh, (((1,), (1,)), ((), ())),
                preferred_element_type=jnp.float32) + bhh[None, :]
            git = gi[:, t, :]
            ir = git[:, :HIDDEN]
            iz = git[:, HIDDEN:2 * HIDDEN]
            inn = git[:, 2 * HIDDEN:]
            hr = gh[:, :HIDDEN]
            hz = gh[:, HIDDEN:2 * HIDDEN]
            hn = gh[:, 2 * HIDDEN:]
            r = jax.nn.sigmoid(ir + hr)
            z = jax.nn.sigmoid(iz + hz)
            n = jnp.tanh(inn + r * hn)
            h = (1.0 - z) * n + z * h
            ys.append(h)
        return jnp.stack(ys, axis=1), h

    y0, _ = run_layer(x, wih0_ref[:], whh0_ref[:], bih0_ref[:], bhh0_ref[:])
    _, h_last = run_layer(y0, wih1_ref[:], whh1_ref[:], bih1_ref[:], bhh1_ref[:])
    out = jax.lax.dot_general(
        h_last, headw_ref[:], (((1,), (0,)), ((), ())),
        preferred_element_type=jnp.float32) + headb_ref[:][None, :]
    out_ref[:] = out


def _gru_head(h_seq, p):
    NT = h_seq.shape[0]
    grid = NT // GRU_BLOCK

    def w_vmem(shape):
        return pl.BlockSpec(shape, lambda i: (0,) * len(shape),
                            memory_space=pltpu.VMEM)

    return pl.pallas_call(
        _gru_head_body,
        grid=(grid,),
        in_specs=[
            pl.BlockSpec((GRU_BLOCK, T, HIDDEN), lambda i: (i, 0, 0)),
            w_vmem((3 * HIDDEN, HIDDEN)), w_vmem((3 * HIDDEN, HIDDEN)),
            w_vmem((3 * HIDDEN,)), w_vmem((3 * HIDDEN,)),
            w_vmem((3 * HIDDEN, HIDDEN)), w_vmem((3 * HIDDEN, HIDDEN)),
            w_vmem((3 * HIDDEN,)), w_vmem((3 * HIDDEN,)),
            w_vmem((HIDDEN, T_FUTURE)), w_vmem((T_FUTURE,)),
        ],
        out_specs=pl.BlockSpec((GRU_BLOCK, T_FUTURE), lambda i: (i, 0)),
        out_shape=jax.ShapeDtypeStruct((NT, T_FUTURE), jnp.float32),
    )(h_seq, p['gru_Wih0'], p['gru_Whh0'], p['gru_bih0'], p['gru_bhh0'],
      p['gru_Wih1'], p['gru_Whh1'], p['gru_bih1'], p['gru_bhh1'],
      p['head_W'], p['head_b'])


def _layer_norm(x, g, b):
    mu = jnp.mean(x, axis=-1, keepdims=True)
    var = jnp.mean((x - mu) ** 2, axis=-1, keepdims=True)
    return (x - mu) / jnp.sqrt(var + 1e-5) * g + b


def _gat(h, src, dst, edge_attr, lin_W, bias, att_src, att_dst, att_edge,
         edge_W, n_nodes):
    ones = jnp.ones((dst.shape[0],), jnp.float32)
    deg = jax.ops.segment_sum(ones, dst, num_segments=n_nodes)
    loop_attr = (jax.ops.segment_sum(edge_attr, dst, num_segments=n_nodes)
                 / jnp.maximum(deg, 1.0)[:, None])
    loop = jnp.arange(n_nodes, dtype=src.dtype)
    src_a = jnp.concatenate([src, loop])
    dst_a = jnp.concatenate([dst, loop])
    ea = jnp.concatenate([edge_attr, loop_attr], axis=0)
    x = (h @ lin_W).reshape(n_nodes, HEADS, OUT_CH)
    a_src = jnp.sum(x * att_src, axis=-1)
    a_dst = jnp.sum(x * att_dst, axis=-1)
    ef = (ea @ edge_W).reshape(-1, HEADS, OUT_CH)
    a_edge = jnp.sum(ef * att_edge, axis=-1)
    alpha = a_src[src_a] + a_dst[dst_a] + a_edge
    alpha = jax.nn.leaky_relu(alpha, 0.2)
    amax = jax.ops.segment_max(alpha, dst_a, num_segments=n_nodes)
    ex = jnp.exp(alpha - amax[dst_a])
    den = jax.ops.segment_sum(ex, dst_a, num_segments=n_nodes)
    w = ex / (den[dst_a] + 1e-16)
    out = jax.ops.segment_sum(x[src_a] * w[:, :, None], dst_a,
                              num_segments=n_nodes)
    return out.reshape(n_nodes, HIDDEN) + bias


def kernel(x_seq, edge_index, edge_weight, params):
    p = params
    B, N, Tdim, C = x_seq.shape
    NT = B * N
    offsets = (jnp.arange(B, dtype=edge_index.dtype) * N)[:, None, None]
    batch_ei = (edge_index[None, :, :] + offsets).reshape(2, -1)
    src, dst = batch_ei[0], batch_ei[1]
    ea = jnp.tile(edge_weight, B)[:, None]
    embs = []
    for t in range(Tdim):
        x_t = x_seq[:, :, t, :].reshape(NT, C)
        skip = x_t @ p['skip_W'] + p['skip_b']
        h = x_t
        for i in range(2):
            h_in = h
            g = str(i)
            h = _gat(h, src, dst, ea, p['g' + g + '_lin_W'],
                     p['g' + g + '_bias'], p['g' + g + '_att_src'],
                     p['g' + g + '_att_dst'], p['g' + g + '_att_edge'],
                     p['g' + g + '_edge_W'], NT)
            h = _layer_norm(h, p['n' + g + '_g'], p['n' + g + '_b'])
            h = h + (skip if i == 0 else h_in)
            h = jax.nn.relu(h)
        embs.append(h)
    h_seq = jnp.stack(embs, axis=1)
    out = _gru_head(h_seq, p)
    return out.reshape(B, N, T_FUTURE)


# speedup vs baseline: 31.8224x; 31.8224x over previous
import functools

import jax
import jax.numpy as jnp
from jax import lax
from jax.experimental import pallas as pl
from jax.experimental.pallas import tpu as pltpu
from jax.experimental.pallas import tpu_sc as plsc

HIDDEN = 64
HEADS = 4
OUT_CH = 16
T_FUTURE = 14
T = 12

NWORK = 32
RNG = 640
NP = NWORK * RNG
CH = 128
TRASH = RNG
CP = 8

BLK = 1024
GRU_BLOCK = 512

_f32 = jnp.float32
_i32 = jnp.int32


def _prep_body(xp_ref, w0_ref, as0_ref, ad0_ref, skw_ref, skb_ref,
               xa_ref, a0_ref, b0_ref, sk_ref):
    x = xp_ref[0]
    x0 = jax.lax.dot_general(x, w0_ref[:], (((1,), (0,)), ((), ())),
                             preferred_element_type=_f32)
    a0 = jax.lax.dot_general(x0, as0_ref[:], (((1,), (0,)), ((), ())),
                             preferred_element_type=_f32)
    xa_ref[0, :, 0:HIDDEN] = x0
    xa_ref[0, :, HIDDEN:HIDDEN + HEADS] = a0
    xa_ref[0, :, HIDDEN + HEADS:] = jnp.zeros(
        (x.shape[0], 128 - HIDDEN - HEADS), _f32)
    a0_ref[0] = a0
    b0_ref[0] = jax.lax.dot_general(x0, ad0_ref[:], (((1,), (0,)), ((), ())),
                                    preferred_element_type=_f32)
    sk_ref[0] = jax.lax.dot_general(x, skw_ref[:], (((1,), (0,)), ((), ())),
                                    preferred_element_type=_f32) \
        + skb_ref[:][None, :]


def _prep(xp, w0, as0, ad0, skw, skb):
    def wv(shape):
        return pl.BlockSpec(shape, lambda t, i: (0,) * len(shape),
                            memory_space=pltpu.VMEM)
    nb = pl.BlockSpec((1, BLK, CP), lambda t, i: (t, i, 0))
    o128 = pl.BlockSpec((1, BLK, 128), lambda t, i: (t, i, 0))
    o64 = pl.BlockSpec((1, BLK, HIDDEN), lambda t, i: (t, i, 0))
    o4 = pl.BlockSpec((1, BLK, HEADS), lambda t, i: (t, i, 0))
    return pl.pallas_call(
        _prep_body,
        grid=(T, NP // BLK),
        in_specs=[nb, wv((CP, HIDDEN)), wv((HIDDEN, HEADS)),
                  wv((HIDDEN, HEADS)), wv((CP, HIDDEN)), wv((HIDDEN,))],
        out_specs=[o128, o4, o4, o64],
        out_shape=[jax.ShapeDtypeStruct((T, NP, 128), _f32),
                   jax.ShapeDtypeStruct((T, NP, HEADS), _f32),
                   jax.ShapeDtypeStruct((T, NP, HEADS), _f32),
                   jax.ShapeDtypeStruct((T, NP, HIDDEN), _f32)],
    )(xp, w0, as0, ad0, skw, skb)


def _ln(x, g, b):
    mu = jnp.mean(x, axis=-1, keepdims=True)
    var = jnp.mean((x - mu) ** 2, axis=-1, keepdims=True)
    return (x - mu) / jnp.sqrt(var + 1e-5) * g + b


def _mid_body(num_ref, den_ref, sk_ref, expm_ref, bias_ref, ng_ref, nb_ref,
              w1_ref, as1_ref, ad1_ref, xa_ref, a1_ref, b1_ref, h1_ref):
    den64 = jax.lax.dot_general(den_ref[0], expm_ref[:],
                                (((1,), (0,)), ((), ())),
                                preferred_element_type=_f32)
    gat = num_ref[0] / jnp.maximum(den64, 1e-30) + bias_ref[:][None, :]
    h1 = jax.nn.relu(_ln(gat, ng_ref[:][None, :], nb_ref[:][None, :])
                     + sk_ref[0])
    h1_ref[0] = h1
    x1 = jax.lax.dot_general(h1, w1_ref[:], (((1,), (0,)), ((), ())),
                             preferred_element_type=_f32)
    a1 = jax.lax.dot_general(x1, as1_ref[:], (((1,), (0,)), ((), ())),
                             preferred_element_type=_f32)
    xa_ref[0, :, 0:HIDDEN] = x1
    xa_ref[0, :, HIDDEN:HIDDEN + HEADS] = a1
    xa_ref[0, :, HIDDEN + HEADS:] = jnp.zeros(
        (x1.shape[0], 128 - HIDDEN - HEADS), _f32)
    a1_ref[0] = a1
    b1_ref[0] = jax.lax.dot_general(x1, ad1_ref[:], (((1,), (0,)), ((), ())),
                                    preferred_element_type=_f32)


def _mid(num0, den0, sk, expm, bias0, ng, nbp, w1, as1, ad1):
    def wv(shape):
        return pl.BlockSpec(shape, lambda t, i: (0,) * len(shape),
                            memory_space=pltpu.VMEM)
    o64 = pl.BlockSpec((1, BLK, HIDDEN), lambda t, i: (t, i, 0))
    o4 = pl.BlockSpec((1, BLK, HEADS), lambda t, i: (t, i, 0))
    return pl.pallas_call(
        _mid_body,
        grid=(T, NP // BLK),
        in_specs=[o64, o4, o64, wv((HEADS, HIDDEN)), wv((HIDDEN,)),
                  wv((HIDDEN,)), wv((HIDDEN,)), wv((HIDDEN, HIDDEN)),
                  wv((HIDDEN, HEADS)), wv((HIDDEN, HEADS))],
        out_specs=[pl.BlockSpec((1, BLK, 128), lambda t, i: (t, i, 0)),
                   o4, o4, o64],
        out_shape=[jax.ShapeDtypeStruct((T, NP, 128), _f32),
                   jax.ShapeDtypeStruct((T, NP, HEADS), _f32),
                   jax.ShapeDtypeStruct((T, NP, HEADS), _f32),
                   jax.ShapeDtypeStruct((T, NP, HIDDEN), _f32)],
    )(num0, den0, sk, expm, bias0, ng, nbp, w1, as1, ad1)


def _gru_body(num_ref, den_ref, h1_ref, expm_ref, bias_ref, ng_ref, nb_ref,
              wih0_ref, whh0_ref, bih0_ref, bhh0_ref,
              wih1_ref, whh1_ref, bih1_ref, bhh1_ref,
              headw_ref, headb_ref, out_ref):
    expm = expm_ref[:]
    bias = bias_ref[:][None, :]
    ng = ng_ref[:][None, :]
    nbv = nb_ref[:][None, :]
    xs = []
    for t in range(T):
        den64 = jax.lax.dot_general(den_ref[t], expm,
                                    (((1,), (0,)), ((), ())),
                                    preferred_element_type=_f32)
        gat = num_ref[t] / jnp.maximum(den64, 1e-30) + bias
        xs.append(jax.nn.relu(_ln(gat, ng, nbv) + h1_ref[t]))

    def run_layer(x_list, wih, whh, bih, bhh):
        R = x_list[0].shape[0]
        h = jnp.zeros((R, HIDDEN), _f32)
        ys = []
        for t in range(T):
            gi = jax.lax.dot_general(x_list[t], wih, (((1,), (1,)), ((), ())),
                                     preferred_element_type=_f32) \
                + bih[None, :]
            gh = jax.lax.dot_general(h, whh, (((1,), (1,)), ((), ())),
                                     preferred_element_type=_f32) \
                + bhh[None, :]
            r = jax.nn.sigmoid(gi[:, :HIDDEN] + gh[:, :HIDDEN])
            z = jax.nn.sigmoid(gi[:, HIDDEN:2 * HIDDEN]
                               + gh[:, HIDDEN:2 * HIDDEN])
            n = jnp.tanh(gi[:, 2 * HIDDEN:] + r * gh[:, 2 * HIDDEN:])
            h = (1.0 - z) * n + z * h
            ys.append(h)
        return ys, h

    y0, _ = run_layer(xs, wih0_ref[:], whh0_ref[:], bih0_ref[:], bhh0_ref[:])
    _, h_last = run_layer(y0, wih1_ref[:], whh1_ref[:], bih1_ref[:],
                          bhh1_ref[:])
    out_ref[:] = jax.lax.dot_general(
        h_last, headw_ref[:], (((1,), (0,)), ((), ())),
        preferred_element_type=_f32) + headb_ref[:][None, :]


def _gru(num1, den1, h1res, expm, bias1, ng, nbp, p):
    def wv(shape):
        return pl.BlockSpec(shape, lambda i: (0,) * len(shape),
                            memory_space=pltpu.VMEM)
    return pl.pallas_call(
        _gru_body,
        grid=(NP // GRU_BLOCK,),
        in_specs=[
            pl.BlockSpec((T, GRU_BLOCK, HIDDEN), lambda i: (0, i, 0)),
            pl.BlockSpec((T, GRU_BLOCK, HEADS), lambda i: (0, i, 0)),
            pl.BlockSpec((T, GRU_BLOCK, HIDDEN), lambda i: (0, i, 0)),
            wv((HEADS, HIDDEN)), wv((HIDDEN,)), wv((HIDDEN,)), wv((HIDDEN,)),
            wv((3 * HIDDEN, HIDDEN)), wv((3 * HIDDEN, HIDDEN)),
            wv((3 * HIDDEN,)), wv((3 * HIDDEN,)),
            wv((3 * HIDDEN, HIDDEN)), wv((3 * HIDDEN, HIDDEN)),
            wv((3 * HIDDEN,)), wv((3 * HIDDEN,)),
            wv((HIDDEN, T_FUTURE)), wv((T_FUTURE,)),
        ],
        out_specs=pl.BlockSpec((GRU_BLOCK, T_FUTURE), lambda i: (i, 0)),
        out_shape=jax.ShapeDtypeStruct((NP, T_FUTURE), _f32),
    )(num1, den1, h1res, expm, bias1, ng, nbp,
      p['gru_Wih0'], p['gru_Whh0'], p['gru_bih0'], p['gru_bhh0'],
      p['gru_Wih1'], p['gru_Whh1'], p['gru_bih1'], p['gru_bhh1'],
      p['head_W'], p['head_b'])


def _sc_loop_attr(dl_p, v_p, st_pad):
    mesh = plsc.VectorSubcoreMesh(core_axis_name="c", subcore_axis_name="s")

    @functools.partial(
        pl.kernel,
        out_type=jax.ShapeDtypeStruct((NP,), _f32),
        mesh=mesh,
        compiler_params=pltpu.CompilerParams(needs_layout_passes=False),
        scratch_types=[
            pltpu.VMEM((RNG + 16,), _f32),
            pltpu.VMEM((RNG + 16,), _f32),
            pltpu.VMEM((CH,), _i32),
            pltpu.VMEM((CH,), _f32),
            pltpu.VMEM((48,), _i32),
        ],
    )
    def k(dl_h, v_h, st_h, lat_h, wsum, cnt, didx, vv, st_loc):
        kid = lax.axis_index("s") * 2 + lax.axis_index("c")
        own = kid * RNG
        pltpu.sync_copy(st_h, st_loc.at[pl.ds(0, 40)])
        zv = jnp.zeros((16,), _f32)
        ones = jnp.full((16,), 1.0, _f32)

        def zb(i, _):
            wsum[pl.ds(i * 16, 16)] = zv
            cnt[pl.ds(i * 16, 16)] = zv
            return 0
        lax.fori_loop(0, (RNG + 16) // 16, zb, 0)
        stv = st_loc[pl.ds(kid, 16)]
        e0 = stv[0]
        e1 = stv[1]
        nch = (e1 - e0) // CH

        def chunk(i, _):
            cbase = pl.multiple_of(e0 + i * CH, CH)
            pltpu.sync_copy(dl_h.at[pl.ds(cbase, CH)], didx)
            pltpu.sync_copy(v_h.at[pl.ds(cbase, CH)], vv)

            def grp(g, _):
                d16 = didx[pl.ds(g * 16, 16)]
                v16 = vv[pl.ds(g * 16, 16)]
                plsc.addupdate_scatter(wsum, [d16], v16)
                plsc.addupdate_scatter(cnt, [d16], ones)
                return 0
            lax.fori_loop(0, CH // 16, grp, 0)
            return 0
        lax.fori_loop(0, nch, chunk, 0)

        def fin(j, _):
            w = wsum[pl.ds(j * 16, 16)]
            c = cnt[pl.ds(j * 16, 16)]
            wsum[pl.ds(j * 16, 16)] = w / jnp.maximum(c, 1.0)
            return 0
        lax.fori_loop(0, RNG // 16, fin, 0)
        pltpu.sync_copy(wsum.at[pl.ds(0, RNG)],
                        lat_h.at[pl.ds(pl.multiple_of(own, CH), RNG)])

    return k(dl_p, v_p, st_pad)


def _sc_edge(xa2, btab, mtab, cvec, lat, src_p, dl_p, v_p, st_pad, nt):
    mesh = plsc.VectorSubcoreMesh(core_axis_name="c", subcore_axis_name="s")

    @functools.partial(
        pl.kernel,
        out_type=[jax.ShapeDtypeStruct((T * NP * HIDDEN,), _f32),
                  jax.ShapeDtypeStruct((T * NP * HEADS,), _f32)],
        mesh=mesh,
        compiler_params=pltpu.CompilerParams(needs_layout_passes=False),
        scratch_types=[
            pltpu.VMEM(((RNG + 8) * HIDDEN,), _f32),
            pltpu.VMEM(((RNG + 8) * HEADS,), _f32),
            pltpu.VMEM((RNG * HEADS,), _f32),
            pltpu.VMEM((RNG,), _f32),
            pltpu.VMEM((16,), _f32),
            pltpu.VMEM((16,), _f32),
            pltpu.VMEM((48,), _i32),
            pltpu.VMEM((CH,), _i32),
            pltpu.VMEM((CH,), _i32),
            pltpu.VMEM((CH,), _f32),
            pltpu.VMEM((CH,), _i32),
            pltpu.VMEM((CH, 128), _f32),
            pltpu.VMEM((CH * HEADS,), _f32),
            pltpu.SemaphoreType.DMA,
        ],
    )
    def k(xa_h, bt_h, mt_h, cv_h, lat_h, src_h, dl_h, v_h, st_h,
          num_h, den_h,
          num_acc, den_acc, b_loc, la_loc, m_loc, c_loc, st_loc,
          sidx, didx, vv, gidx, xrows, pbuf, sem):
        kid = lax.axis_index("s") * 2 + lax.axis_index("c")
        own = pl.multiple_of(kid * RNG, CH)
        own_cnt = jnp.minimum(RNG, nt - own)
        pltpu.sync_copy(st_h, st_loc.at[pl.ds(0, 40)])
        pltpu.sync_copy(cv_h, c_loc)
        pltpu.sync_copy(lat_h.at[pl.ds(own, RNG)], la_loc)
        iota = lax.iota(_i32, 16)
        div4 = jnp.right_shift(iota, 2)
        mod4 = jnp.bitwise_and(iota, 3)
        zv = jnp.zeros((16,), _f32)
        c16 = c_loc[:]
        stv = st_loc[pl.ds(kid, 16)]
        e0 = stv[0]
        e1 = stv[1]
        nch = (e1 - e0) // CH

        def groups(rdidx, m16):
            def grp(g, _):
                evec = g * 4 + div4
                d4 = plsc.load_gather(rdidx, [evec])
                bidx = d4 * 4 + mod4
                b4 = plsc.load_gather(b_loc, [bidx])
                a4 = plsc.load_gather(xrows, [evec, HIDDEN + mod4])
                v4 = plsc.load_gather(vv, [evec])
                z = a4 + b4 + c16 * v4
                alpha = jnp.maximum(z, z * 0.2)
                p = jnp.exp(alpha - m16)
                plsc.addupdate_scatter(den_acc, [bidx], p)
                pbuf[pl.ds(g * 16, 16)] = p
                for j in range(4):
                    e = g * 4 + j
                    dlv = plsc.load_gather(rdidx, [jnp.broadcast_to(e, (16,))])
                    base = dlv * HIDDEN
                    for q in range(4):
                        pqv = plsc.load_gather(
                            pbuf, [jnp.broadcast_to(e * 4 + q, (16,))])
                        xr = plsc.load_gather(
                            xrows, [jnp.broadcast_to(e, (16,)),
                                    q * 16 + iota])
                        idxv = base + (q * 16 + iota)
                        plsc.addupdate_scatter(num_acc, [idxv], xr * pqv)
                return 0
            lax.fori_loop(0, CH // 4, grp, 0)

        def per_t(t, _):
            pltpu.sync_copy(mt_h.at[t], m_loc)
            pltpu.sync_copy(
                bt_h.at[pl.ds(pl.multiple_of((t * NP + own) * HEADS, CH),
                              RNG * HEADS)],
                b_loc)
            m16 = m_loc[:]

            def zb(i, _):
                num_acc[pl.ds(i * 16, 16)] = zv
                return 0
            lax.fori_loop(0, (RNG + 8) * HIDDEN // 16, zb, 0)

            def zd(i, _):
                den_acc[pl.ds(i * 16, 16)] = zv
                return 0
            lax.fori_loop(0, (RNG + 8) * HEADS // 16, zd, 0)

            tnp = t * NP

            def chunk(i, _):
                cbase = pl.multiple_of(e0 + i * CH, CH)
                pltpu.sync_copy(src_h.at[pl.ds(cbase, CH)], sidx)
                pltpu.sync_copy(dl_h.at[pl.ds(cbase, CH)], didx)
                pltpu.sync_copy(v_h.at[pl.ds(cbase, CH)], vv)

                def addt(j, _):
                    gidx[pl.ds(j * 16, 16)] = sidx[pl.ds(j * 16, 16)] + tnp
                    return 0
                lax.fori_loop(0, CH // 16, addt, 0)
                pltpu.async_copy(xa_h.at[gidx], xrows, sem).wait()
                groups(didx, m16)
                return 0
            lax.fori_loop(0, nch, chunk, 0)

            def self_chunk(i, _):
                cbase = i * CH

                def mk(j, _):
                    ramp = iota + (cbase + j * 16)
                    didx[pl.ds(j * 16, 16)] = jnp.where(
                        ramp < own_cnt, ramp, TRASH)
                    gidx[pl.ds(j * 16, 16)] = ramp + (tnp + own)
                    vv[pl.ds(j * 16, 16)] = la_loc[pl.ds(cbase + j * 16, 16)]
                    return 0
                lax.fori_loop(0, CH // 16, mk, 0)
                pltpu.async_copy(xa_h.at[gidx], xrows, sem).wait()
                groups(didx, m16)
                return 0
            lax.fori_loop(0, RNG // CH, self_chunk, 0)

            pltpu.sync_copy(
                num_acc.at[pl.ds(0, RNG * HIDDEN)],
                num_h.at[pl.ds(pl.multiple_of((tnp + own) * HIDDEN, CH),
                               RNG * HIDDEN)])
            pltpu.sync_copy(
                den_acc.at[pl.ds(0, RNG * HEADS)],
                den_h.at[pl.ds(pl.multiple_of((tnp + own) * HEADS, CH),
                               RNG * HEADS)])
            return 0
        lax.fori_loop(0, T, per_t, 0)

    return k(xa2, btab, mtab, cvec, lat, src_p, dl_p, v_p, st_pad)


def _att_mat(att):
    return (att[:, :, None] * jnp.eye(HEADS, dtype=_f32)[:, None, :]
            ).reshape(HIDDEN, HEADS)


def kernel(x_seq, edge_index, edge_weight, params):
    p = params
    B, N, Tdim, C = x_seq.shape
    NT = B * N
    E = edge_weight.shape[0]
    EB = B * E
    EBP = EB + NWORK * CH

    offsets = (jnp.arange(B, dtype=edge_index.dtype) * N)[:, None, None]
    batch_ei = (edge_index[None, :, :] + offsets).reshape(2, -1)
    src = batch_ei[0].astype(_i32)
    dst = batch_ei[1].astype(_i32)
    ew = jnp.tile(edge_weight, B)
    owner = dst // RNG
    perm = jnp.argsort(owner, stable=True)
    os_ = owner[perm]
    counts = jnp.bincount(owner, length=NWORK)
    starts = jnp.concatenate([jnp.zeros((1,), _i32),
                              jnp.cumsum(counts).astype(_i32)])
    pcounts = ((counts + CH - 1) // CH) * CH
    pstarts = jnp.concatenate([jnp.zeros((1,), _i32),
                               jnp.cumsum(pcounts).astype(_i32)])
    rank = jnp.arange(EB, dtype=_i32) - starts[os_]
    newpos = pstarts[os_] + rank
    src_p = jnp.zeros((EBP,), _i32).at[newpos].set(src[perm])
    dl_p = jnp.full((EBP,), TRASH, _i32).at[newpos].set(
        (dst % RNG)[perm])
    v_p = jnp.zeros((EBP,), _f32).at[newpos].set(ew[perm])
    st_pad = jnp.zeros((40,), _i32).at[:NWORK + 1].set(pstarts)

    w0 = jnp.zeros((CP, HIDDEN), _f32).at[:C].set(p['g0_lin_W'])
    skw = jnp.zeros((CP, HIDDEN), _f32).at[:C].set(p['skip_W'])
    as0 = _att_mat(p['g0_att_src'])
    ad0 = _att_mat(p['g0_att_dst'])
    as1 = _att_mat(p['g1_att_src'])
    ad1 = _att_mat(p['g1_att_dst'])
    c0 = jnp.sum(p['g0_edge_W'].reshape(HEADS, OUT_CH) * p['g0_att_edge'],
                 axis=1)
    c1 = jnp.sum(p['g1_edge_W'].reshape(HEADS, OUT_CH) * p['g1_att_edge'],
                 axis=1)
    cvec0 = jnp.tile(c0, 4)
    cvec1 = jnp.tile(c1, 4)
    expm = jnp.repeat(jnp.eye(HEADS, dtype=_f32), OUT_CH, axis=1)
    vmax = jnp.max(edge_weight)

    xp = jnp.transpose(x_seq, (2, 0, 1, 3)).reshape(Tdim, NT, C)
    xp = jnp.pad(xp, ((0, 0), (0, NP - NT), (0, CP - C)))
    xa0, a0, b0, sk = _prep(xp, w0, as0, ad0, skw, p['skip_b'])

    lat = _sc_loop_attr(dl_p, v_p, st_pad)

    m0 = (jnp.max(a0, axis=(1,)) + jnp.max(b0, axis=(1,))
          + jnp.maximum(c0 * vmax, 0.0)[None, :])
    mtab0 = jnp.tile(m0, (1, 4))
    num0, den0 = _sc_edge(xa0.reshape(T * NP, 128),
                          b0.reshape(-1), mtab0, cvec0, lat,
                          src_p, dl_p, v_p, st_pad, NT)
    num0 = num0.reshape(T, NP, HIDDEN)
    den0 = den0.reshape(T, NP, HEADS)

    xa1, a1, b1, h1 = _mid(num0, den0, sk, expm, p['g0_bias'],
                           p['n0_g'], p['n0_b'], p['g1_lin_W'], as1, ad1)

    m1 = (jnp.max(a1, axis=(1,)) + jnp.max(b1, axis=(1,))
          + jnp.maximum(c1 * vmax, 0.0)[None, :])
    mtab1 = jnp.tile(m1, (1, 4))
    num1, den1 = _sc_edge(xa1.reshape(T * NP, 128),
                          b1.reshape(-1), mtab1, cvec1, lat,
                          src_p, dl_p, v_p, st_pad, NT)
    num1 = num1.reshape(T, NP, HIDDEN)
    den1 = den1.reshape(T, NP, HEADS)

    out = _gru(num1, den1, h1, expm, p['g1_bias'], p['n1_g'], p['n1_b'], p)
    return out[:NT].reshape(B, N, T_FUTURE)
